# baseline (speedup 1.0000x reference)
import jax
import jax.numpy as jnp
from jax.experimental import pallas as pl
from jax.experimental.pallas import tpu as pltpu

N = 4096
D = 512
BJ = 256
NJ = N // BJ
NI_PROJ = 8
NEG_SLOPE = 0.2
LOG2E = 1.4426950408889634
NEG_BIG = -1e30


def _lrelu(x):
    return jnp.maximum(x, NEG_SLOPE * x)


def _proj_body(x_ref, w_ref, asrc_ref, adst_ref,
               xp_ref, as_ref, ad_ref, gmax_ref):
    i = pl.program_id(0)
    xp = jax.lax.dot_general(
        x_ref[...].astype(jnp.bfloat16), w_ref[...].astype(jnp.bfloat16),
        (((1,), (0,)), ((), ())), preferred_element_type=jnp.float32)
    xp_ref[...] = xp.astype(jnp.bfloat16)
    as_blk = jax.lax.dot_general(
        xp, asrc_ref[...], (((1,), (0,)), ((), ())),
        preferred_element_type=jnp.float32)
    as_ref[...] = as_blk
    ad_ref[...] = jax.lax.dot_general(
        xp, adst_ref[...], (((1,), (0,)), ((), ())),
        preferred_element_type=jnp.float32)

    @pl.when(i == 0)
    def _():
        gmax_ref[...] = jnp.full((1, 1), NEG_BIG, jnp.float32)

    gmax_ref[...] = jnp.maximum(gmax_ref[...], jnp.max(as_blk))


def _agg_body(a_ref, xp_ref, as_ref, ad_ref, gmax_ref, bias_ref, out_ref):
    q = gmax_ref[0, 0] + ad_ref[...]
    m2 = _lrelu(q)
    z = as_ref[...] + ad_ref[...]
    e2 = _lrelu(z)
    p = (a_ref[...] * jnp.exp2(e2 - m2)).astype(jnp.bfloat16)
    out = jax.lax.dot_general(
        p, xp_ref[...], (((0,), (0,)), ((), ())),
        preferred_element_type=jnp.float32)
    s = jax.lax.dot_general(
        p, jnp.ones((N, 1), jnp.bfloat16), (((0,), (0,)), ((), ())),
        preferred_element_type=jnp.float32)
    s_safe = jnp.where(s > 0.0, s, 1.0)
    out_ref[...] = jnp.maximum(out / s_safe + bias_ref[...], 0.0)


@jax.jit
def kernel(A, X, W, a_src, a_dst, bias):
    d_in = X.shape[1]
    bi = N // NI_PROJ
    xp, as_col, ad_col, gmax = pl.pallas_call(
        _proj_body,
        grid=(NI_PROJ,),
        in_specs=[
            pl.BlockSpec((bi, d_in), lambda i: (i, 0)),
            pl.BlockSpec((d_in, D), lambda i: (0, 0)),
            pl.BlockSpec((D, 1), lambda i: (0, 0)),
            pl.BlockSpec((D, 1), lambda i: (0, 0)),
        ],
        out_specs=[
            pl.BlockSpec((bi, D), lambda i: (i, 0)),
            pl.BlockSpec((bi, 1), lambda i: (i, 0)),
            pl.BlockSpec((bi, 1), lambda i: (i, 0)),
            pl.BlockSpec((1, 1), lambda i: (0, 0)),
        ],
        out_shape=[
            jax.ShapeDtypeStruct((N, D), jnp.bfloat16),
            jax.ShapeDtypeStruct((N, 1), jnp.float32),
            jax.ShapeDtypeStruct((N, 1), jnp.float32),
            jax.ShapeDtypeStruct((1, 1), jnp.float32),
        ],
        compiler_params=pltpu.CompilerParams(
            dimension_semantics=("arbitrary",)),
    )(X, W, (a_src * LOG2E).reshape(D, 1), (a_dst * LOG2E).reshape(D, 1))

    ad_row = ad_col.reshape(1, N)

    out = pl.pallas_call(
        _agg_body,
        grid=(NJ,),
        in_specs=[
            pl.BlockSpec((N, BJ), lambda j: (0, j)),
            pl.BlockSpec((N, D), lambda j: (0, 0)),
            pl.BlockSpec((N, 1), lambda j: (0, 0)),
            pl.BlockSpec((1, BJ), lambda j: (0, j)),
            pl.BlockSpec((1, 1), lambda j: (0, 0)),
            pl.BlockSpec((1, D), lambda j: (0, 0)),
        ],
        out_specs=pl.BlockSpec((BJ, D), lambda j: (j, 0)),
        out_shape=jax.ShapeDtypeStruct((N, D), jnp.float32),
        compiler_params=pltpu.CompilerParams(
            dimension_semantics=("arbitrary",)),
    )(A, xp, as_col, ad_row, gmax, bias.reshape(1, D))

    return out


# speedup vs baseline: 1.1685x; 1.1685x over previous
import jax
import jax.numpy as jnp
from jax.experimental import pallas as pl
from jax.experimental.pallas import tpu as pltpu

N = 4096
D = 512
BJ = 256
NJ = N // BJ
NI_PROJ = 8
NEG_SLOPE = 0.2
LOG2E = 1.4426950408889634
NEG_BIG = -1e30


def _lrelu(x):
    return jnp.maximum(x, NEG_SLOPE * x)


def _proj_body(x_ref, w_ref, asrc_ref, adst_ref,
               xp_ref, as_ref, ad_ref, gmax_ref):
    i = pl.program_id(0)
    xp = jax.lax.dot_general(
        x_ref[...].astype(jnp.bfloat16), w_ref[...].astype(jnp.bfloat16),
        (((1,), (0,)), ((), ())), preferred_element_type=jnp.float32)
    xp_ref[...] = xp
    as_blk = jax.lax.dot_general(
        xp, asrc_ref[...], (((1,), (0,)), ((), ())),
        preferred_element_type=jnp.float32)
    as_ref[...] = as_blk
    ad_ref[...] = jax.lax.dot_general(
        xp, adst_ref[...], (((1,), (0,)), ((), ())),
        preferred_element_type=jnp.float32)

    @pl.when(i == 0)
    def _():
        gmax_ref[...] = jnp.full((1, 1), NEG_BIG, jnp.float32)

    gmax_ref[...] = jnp.maximum(gmax_ref[...], jnp.max(as_blk))


def _agg_body(a_ref, xp_ref, as_ref, ad_ref, gmax_ref, bias_ref, out_ref):
    q = gmax_ref[0, 0] + ad_ref[...]
    m2 = _lrelu(q)
    z = as_ref[...] + ad_ref[...]
    e2 = _lrelu(z)
    p = a_ref[...] * jnp.exp2(e2 - m2)
    out = jax.lax.dot_general(
        p, xp_ref[...], (((0,), (0,)), ((), ())),
        preferred_element_type=jnp.float32)
    s = jax.lax.dot_general(
        p, jnp.ones((N, 1), jnp.float32), (((0,), (0,)), ((), ())),
        preferred_element_type=jnp.float32)
    s_safe = jnp.where(s > 0.0, s, 1.0)
    out_ref[...] = jnp.maximum(out / s_safe + bias_ref[...], 0.0)


@jax.jit
def kernel(A, X, W, a_src, a_dst, bias):
    d_in = X.shape[1]
    bi = N // NI_PROJ
    xp, as_col, ad_col, gmax = pl.pallas_call(
        _proj_body,
        grid=(NI_PROJ,),
        in_specs=[
            pl.BlockSpec((bi, d_in), lambda i: (i, 0)),
            pl.BlockSpec((d_in, D), lambda i: (0, 0)),
            pl.BlockSpec((D, 1), lambda i: (0, 0)),
            pl.BlockSpec((D, 1), lambda i: (0, 0)),
        ],
        out_specs=[
            pl.BlockSpec((bi, D), lambda i: (i, 0)),
            pl.BlockSpec((bi, 1), lambda i: (i, 0)),
            pl.BlockSpec((bi, 1), lambda i: (i, 0)),
            pl.BlockSpec((1, 1), lambda i: (0, 0)),
        ],
        out_shape=[
            jax.ShapeDtypeStruct((N, D), jnp.float32),
            jax.ShapeDtypeStruct((N, 1), jnp.float32),
            jax.ShapeDtypeStruct((N, 1), jnp.float32),
            jax.ShapeDtypeStruct((1, 1), jnp.float32),
        ],
        compiler_params=pltpu.CompilerParams(
            dimension_semantics=("arbitrary",)),
    )(X, W, (a_src * LOG2E).reshape(D, 1), (a_dst * LOG2E).reshape(D, 1))

    ad_row = ad_col.reshape(1, N)

    out = pl.pallas_call(
        _agg_body,
        grid=(NJ,),
        in_specs=[
            pl.BlockSpec((N, BJ), lambda j: (0, j)),
            pl.BlockSpec((N, D), lambda j: (0, 0)),
            pl.BlockSpec((N, 1), lambda j: (0, 0)),
            pl.BlockSpec((1, BJ), lambda j: (0, j)),
            pl.BlockSpec((1, 1), lambda j: (0, 0)),
            pl.BlockSpec((1, D), lambda j: (0, 0)),
        ],
        out_specs=pl.BlockSpec((BJ, D), lambda j: (j, 0)),
        out_shape=jax.ShapeDtypeStruct((N, D), jnp.float32),
        compiler_params=pltpu.CompilerParams(
            dimension_semantics=("arbitrary",)),
    )(A, xp, as_col, ad_row, gmax, bias.reshape(1, D))

    return out


# speedup vs baseline: 1.1800x; 1.0099x over previous
import jax
import jax.numpy as jnp
from jax.experimental import pallas as pl
from jax.experimental.pallas import tpu as pltpu

N = 4096
D = 512
BJ = 512
NJ = N // BJ
NI_PROJ = 8
NEG_SLOPE = 0.2
LOG2E = 1.4426950408889634
NEG_BIG = -1e30


def _lrelu(x):
    return jnp.maximum(x, NEG_SLOPE * x)


def _proj_body(x_ref, w_ref, asrc_ref, adst_ref,
               xp_ref, as_ref, ad_ref, gmax_ref):
    i = pl.program_id(0)
    xp = jax.lax.dot_general(
        x_ref[...].astype(jnp.bfloat16), w_ref[...].astype(jnp.bfloat16),
        (((1,), (0,)), ((), ())), preferred_element_type=jnp.float32)
    xp_ref[...] = xp
    as_blk = jax.lax.dot_general(
        xp, asrc_ref[...], (((1,), (0,)), ((), ())),
        preferred_element_type=jnp.float32)
    as_ref[...] = as_blk
    ad_ref[...] = jax.lax.dot_general(
        xp, adst_ref[...], (((1,), (0,)), ((), ())),
        preferred_element_type=jnp.float32)

    @pl.when(i == 0)
    def _():
        gmax_ref[...] = jnp.full((1, 1), NEG_BIG, jnp.float32)

    gmax_ref[...] = jnp.maximum(gmax_ref[...], jnp.max(as_blk))


def _agg_body(a_ref, xp_ref, as_ref, ad_ref, gmax_ref, bias_ref, out_ref):
    q = gmax_ref[0, 0] + ad_ref[...]
    m2 = _lrelu(q)
    z = as_ref[...] + ad_ref[...]
    e2 = _lrelu(z)
    p = a_ref[...] * jnp.exp2(e2 - m2)
    out = jax.lax.dot_general(
        p, xp_ref[...], (((0,), (0,)), ((), ())),
        preferred_element_type=jnp.float32)
    s = jax.lax.dot_general(
        p, jnp.ones((N, 1), jnp.float32), (((0,), (0,)), ((), ())),
        preferred_element_type=jnp.float32)
    s_safe = jnp.where(s > 0.0, s, 1.0)
    out_ref[...] = jnp.maximum(out / s_safe + bias_ref[...], 0.0)


@jax.jit
def kernel(A, X, W, a_src, a_dst, bias):
    d_in = X.shape[1]
    bi = N // NI_PROJ
    xp, as_col, ad_col, gmax = pl.pallas_call(
        _proj_body,
        grid=(NI_PROJ,),
        in_specs=[
            pl.BlockSpec((bi, d_in), lambda i: (i, 0)),
            pl.BlockSpec((d_in, D), lambda i: (0, 0)),
            pl.BlockSpec((D, 1), lambda i: (0, 0)),
            pl.BlockSpec((D, 1), lambda i: (0, 0)),
        ],
        out_specs=[
            pl.BlockSpec((bi, D), lambda i: (i, 0)),
            pl.BlockSpec((bi, 1), lambda i: (i, 0)),
            pl.BlockSpec((bi, 1), lambda i: (i, 0)),
            pl.BlockSpec((1, 1), lambda i: (0, 0)),
        ],
        out_shape=[
            jax.ShapeDtypeStruct((N, D), jnp.float32),
            jax.ShapeDtypeStruct((N, 1), jnp.float32),
            jax.ShapeDtypeStruct((N, 1), jnp.float32),
            jax.ShapeDtypeStruct((1, 1), jnp.float32),
        ],
        compiler_params=pltpu.CompilerParams(
            dimension_semantics=("arbitrary",)),
    )(X, W, (a_src * LOG2E).reshape(D, 1), (a_dst * LOG2E).reshape(D, 1))

    ad_row = ad_col.reshape(1, N)

    out = pl.pallas_call(
        _agg_body,
        grid=(NJ,),
        in_specs=[
            pl.BlockSpec((N, BJ), lambda j: (0, j)),
            pl.BlockSpec((N, D), lambda j: (0, 0)),
            pl.BlockSpec((N, 1), lambda j: (0, 0)),
            pl.BlockSpec((1, BJ), lambda j: (0, j)),
            pl.BlockSpec((1, 1), lambda j: (0, 0)),
            pl.BlockSpec((1, D), lambda j: (0, 0)),
        ],
        out_specs=pl.BlockSpec((BJ, D), lambda j: (j, 0)),
        out_shape=jax.ShapeDtypeStruct((N, D), jnp.float32),
        compiler_params=pltpu.CompilerParams(
            dimension_semantics=("arbitrary",)),
    )(A, xp, as_col, ad_row, gmax, bias.reshape(1, D))

    return out


# speedup vs baseline: 1.1800x; 1.0000x over previous
import jax
import jax.numpy as jnp
from jax.experimental import pallas as pl
from jax.experimental.pallas import tpu as pltpu

N = 4096
D = 512
BJ = 512
NJ = N // BJ
NI_PROJ = 8
NEG_SLOPE = 0.2
LOG2E = 1.4426950408889634
NEG_BIG = -1e30


def _lrelu(x):
    return jnp.maximum(x, NEG_SLOPE * x)


def _proj_body(x_ref, w_ref, asrc_ref, adst_ref,
               xp_ref, as_ref, ad_ref, gmax_ref):
    i = pl.program_id(0)
    xp = jax.lax.dot_general(
        x_ref[...].astype(jnp.bfloat16), w_ref[...].astype(jnp.bfloat16),
        (((1,), (0,)), ((), ())), preferred_element_type=jnp.float32)
    xp_ref[...] = xp
    as_blk = jax.lax.dot_general(
        xp, asrc_ref[...], (((1,), (0,)), ((), ())),
        preferred_element_type=jnp.float32)
    as_ref[...] = as_blk
    ad_ref[...] = jax.lax.dot_general(
        xp, adst_ref[...], (((1,), (0,)), ((), ())),
        preferred_element_type=jnp.float32)

    @pl.when(i == 0)
    def _():
        gmax_ref[...] = jnp.full((1, 1), NEG_BIG, jnp.float32)

    gmax_ref[...] = jnp.maximum(gmax_ref[...], jnp.max(as_blk))


def _agg_body(a_ref, xp_ref, as_ref, ad_ref, gmax_ref, bias_ref, out_ref):
    q = gmax_ref[0, 0] + ad_ref[...]
    m2 = _lrelu(q)
    z = as_ref[...] + ad_ref[...]
    e2 = _lrelu(z)
    p = a_ref[...] * jnp.exp2(e2 - m2)
    out = jax.lax.dot_general(
        p, xp_ref[...], (((0,), (0,)), ((), ())),
        preferred_element_type=jnp.float32)
    s = jax.lax.dot_general(
        p, jnp.ones((N, 1), jnp.float32), (((0,), (0,)), ((), ())),
        preferred_element_type=jnp.float32)
    s_safe = jnp.where(s > 0.0, s, 1.0)
    out_ref[...] = jnp.maximum(out / s_safe + bias_ref[...], 0.0)


@jax.jit
def kernel(A, X, W, a_src, a_dst, bias):
    d_in = X.shape[1]
    bi = N // NI_PROJ
    xp, as_col, ad_col, gmax = pl.pallas_call(
        _proj_body,
        grid=(NI_PROJ,),
        in_specs=[
            pl.BlockSpec((bi, d_in), lambda i: (i, 0)),
            pl.BlockSpec((d_in, D), lambda i: (0, 0)),
            pl.BlockSpec((D, 1), lambda i: (0, 0)),
            pl.BlockSpec((D, 1), lambda i: (0, 0)),
        ],
        out_specs=[
            pl.BlockSpec((bi, D), lambda i: (i, 0)),
            pl.BlockSpec((bi, 1), lambda i: (i, 0)),
            pl.BlockSpec((bi, 1), lambda i: (i, 0)),
            pl.BlockSpec((1, 1), lambda i: (0, 0)),
        ],
        out_shape=[
            jax.ShapeDtypeStruct((N, D), jnp.float32),
            jax.ShapeDtypeStruct((N, 1), jnp.float32),
            jax.ShapeDtypeStruct((N, 1), jnp.float32),
            jax.ShapeDtypeStruct((1, 1), jnp.float32),
        ],
        compiler_params=pltpu.CompilerParams(
            dimension_semantics=("arbitrary",)),
    )(X, W, (a_src * LOG2E).reshape(D, 1), (a_dst * LOG2E).reshape(D, 1))

    ad_row = ad_col.reshape(1, N)

    out = pl.pallas_call(
        _agg_body,
        grid=(NJ,),
        in_specs=[
            pl.BlockSpec((N, BJ), lambda j: (0, j)),
            pl.BlockSpec((N, D), lambda j: (0, 0)),
            pl.BlockSpec((N, 1), lambda j: (0, 0)),
            pl.BlockSpec((1, BJ), lambda j: (0, j)),
            pl.BlockSpec((1, 1), lambda j: (0, 0)),
            pl.BlockSpec((1, D), lambda j: (0, 0)),
        ],
        out_specs=pl.BlockSpec((BJ, D), lambda j: (j, 0)),
        out_shape=jax.ShapeDtypeStruct((N, D), jnp.float32),
        compiler_params=pltpu.CompilerParams(
            dimension_semantics=("parallel",)),
    )(A, xp, as_col, ad_row, gmax, bias.reshape(1, D))

    return out


# speedup vs baseline: 1.3777x; 1.1675x over previous
import jax
import jax.numpy as jnp
from jax.experimental import pallas as pl
from jax.experimental.pallas import tpu as pltpu

N = 4096
D = 512
BJ = 512
NJ = N // BJ
NP = 8
BI = N // NP
NEG_SLOPE = 0.2
LOG2E = 1.4426950408889634
NEG_BIG = -1e30


def _lrelu(x):
    return jnp.maximum(x, NEG_SLOPE * x)


def _body(x_ref, w_ref, asrc_ref, adst_ref, a_ref, bias_ref, out_ref,
          xp_ref, as_ref, ad_ref, gmax_ref):
    t = pl.program_id(0)

    @pl.when(t < NP)
    def _proj():
        i = t
        xp = jax.lax.dot_general(
            x_ref[...].astype(jnp.bfloat16), w_ref[...].astype(jnp.bfloat16),
            (((1,), (0,)), ((), ())), preferred_element_type=jnp.float32)
        xp_ref[pl.ds(i * BI, BI), :] = xp
        as_blk = jax.lax.dot_general(
            xp, asrc_ref[...] * LOG2E, (((1,), (0,)), ((), ())),
            preferred_element_type=jnp.float32)
        as_ref[pl.ds(i * BI, BI), :] = as_blk
        ad_ref[0:1, pl.ds(i * BI, BI)] = jax.lax.dot_general(
            adst_ref[...] * LOG2E, xp, (((0,), (1,)), ((), ())),
            preferred_element_type=jnp.float32)
        prev = jnp.where(i == 0, jnp.full((1, 1), NEG_BIG, jnp.float32),
                         gmax_ref[...])
        gmax_ref[...] = jnp.maximum(prev, jnp.max(as_blk))

    @pl.when(t >= NP)
    def _agg():
        j = t - NP
        ad_row = ad_ref[0:1, pl.ds(j * BJ, BJ)]
        m2 = _lrelu(gmax_ref[0, 0] + ad_row)
        z = as_ref[...] + ad_row
        e2 = _lrelu(z)
        p = a_ref[...] * jnp.exp2(e2 - m2)
        out = jax.lax.dot_general(
            p, xp_ref[...], (((0,), (0,)), ((), ())),
            preferred_element_type=jnp.float32)
        s = jax.lax.dot_general(
            p, jnp.ones((N, 1), jnp.float32), (((0,), (0,)), ((), ())),
            preferred_element_type=jnp.float32)
        s_safe = jnp.where(s > 0.0, s, 1.0)
        out_ref[...] = jnp.maximum(out / s_safe + bias_ref[...], 0.0)


@jax.jit
def kernel(A, X, W, a_src, a_dst, bias):
    d_in = X.shape[1]
    out = pl.pallas_call(
        _body,
        grid=(NP + NJ,),
        in_specs=[
            pl.BlockSpec((BI, d_in), lambda t: (jnp.minimum(t, NP - 1), 0)),
            pl.BlockSpec((d_in, D), lambda t: (0, 0)),
            pl.BlockSpec((D, 1), lambda t: (0, 0)),
            pl.BlockSpec((D, 1), lambda t: (0, 0)),
            pl.BlockSpec((N, BJ), lambda t: (0, jnp.maximum(t - NP, 0))),
            pl.BlockSpec((1, D), lambda t: (0, 0)),
        ],
        out_specs=pl.BlockSpec((BJ, D), lambda t: (jnp.maximum(t - NP, 0), 0)),
        out_shape=jax.ShapeDtypeStruct((N, D), jnp.float32),
        scratch_shapes=[
            pltpu.VMEM((N, D), jnp.float32),
            pltpu.VMEM((N, 1), jnp.float32),
            pltpu.VMEM((1, N), jnp.float32),
            pltpu.VMEM((1, 1), jnp.float32),
        ],
        compiler_params=pltpu.CompilerParams(
            dimension_semantics=("arbitrary",)),
    )(X, W, a_src.reshape(D, 1), a_dst.reshape(D, 1), A, bias.reshape(1, D))

    return out
